# Initial kernel scaffold; baseline (speedup 1.0000x reference)
import jax
import jax.numpy as jnp
from jax.experimental import pallas as pl


def kernel(inventory, node_embeds, W, b):
    raise NotImplementedError("write your pallas kernel here")


# speedup vs baseline: 3.5541x; 3.5541x over previous
import functools

import jax
import jax.numpy as jnp
from jax import lax
from jax.experimental import pallas as pl
from jax.experimental.pallas import tpu as pltpu
from jax.experimental.pallas import tpu_sc as plsc

_SUB = 128
_GROUP = 9


def _gather_call(table, idx3d, n_rows, embed, out_shape):
    info = plsc.get_sparse_core_info()
    nc, ns = info.num_cores, info.num_subcores
    nw = nc * ns
    assert n_rows % (nw * _SUB) == 0
    sub_per_w = n_rows // (nw * _SUB)
    assert sub_per_w % _GROUP == 0
    n_groups = sub_per_w // _GROUP
    rows_per_group = _GROUP * _SUB
    rows_per_w = sub_per_w * _SUB

    mesh = plsc.VectorSubcoreMesh(core_axis_name="c", subcore_axis_name="s")
    out_rows_per_group = rows_per_group * embed // out_shape[1]
    out_rows_per_w = rows_per_w * embed // out_shape[1]

    @functools.partial(
        pl.kernel,
        mesh=mesh,
        compiler_params=pltpu.CompilerParams(use_tc_tiling_on_sc=False),
        out_type=jax.ShapeDtypeStruct(out_shape, jnp.float32),
        scratch_types=[
            pltpu.VMEM((sub_per_w, _SUB), jnp.int32),
            pltpu.VMEM((rows_per_group, embed), jnp.float32),
            pltpu.SemaphoreType.DMA,
        ],
    )
    def gather_kernel(table_hbm, idx_hbm, out_hbm, idx_v, rows_v, gsem):
        wid = lax.axis_index("s") * nc + lax.axis_index("c")
        base_out = wid * out_rows_per_w
        pltpu.sync_copy(idx_hbm.at[wid], idx_v)
        for g in range(n_groups):
            copies = []
            for s in range(_GROUP):
                copies.append(pltpu.async_copy(
                    table_hbm.at[idx_v.at[g * _GROUP + s]],
                    rows_v.at[pl.ds(s * _SUB, _SUB)],
                    gsem,
                ))
            for c in copies:
                c.wait()
            pltpu.sync_copy(
                rows_v,
                out_hbm.at[pl.ds(wid * rows_per_w + g * rows_per_group,
                                 rows_per_group)],
            )

    return gather_kernel(table, idx3d)


def _mm_body(x_ref, w_ref, b_ref, o_ref):
    o_ref[...] = (
        jnp.dot(x_ref[...], w_ref[...], preferred_element_type=jnp.float32)
        + b_ref[...]
    )


def _matmul(x, wt, b2, block_m):
    m, k = x.shape
    _, n = wt.shape
    return pl.pallas_call(
        _mm_body,
        grid=(m // block_m,),
        in_specs=[
            pl.BlockSpec((block_m, k), lambda i: (i, 0)),
            pl.BlockSpec((k, n), lambda i: (0, 0)),
            pl.BlockSpec((1, n), lambda i: (0, 0)),
        ],
        out_specs=pl.BlockSpec((block_m, n), lambda i: (i, 0)),
        out_shape=jax.ShapeDtypeStruct((m, n), jnp.float32),
    )(x, wt, b2)


def kernel(inventory, node_embeds, W, b):
    batch, n = inventory.shape
    vocab, embed = node_embeds.shape
    out_dim = W.shape[0]
    n_rows = batch * n
    feat = n * embed
    info = plsc.get_sparse_core_info()
    nw = info.num_cores * info.num_subcores
    idx3d = inventory.reshape(nw, n_rows // (nw * _SUB), _SUB)
    gathered = _gather_call(node_embeds, idx3d, n_rows, embed,
                            out_shape=(n_rows, embed))
    x2 = gathered.reshape(batch // 2, 2 * feat)
    wt = W.T
    zeros = jnp.zeros_like(wt)
    w2 = jnp.block([[wt, zeros], [zeros, wt]])
    b2 = jnp.concatenate([b, b]).reshape(1, 2 * out_dim)
    out2 = _matmul(x2, w2, b2, block_m=512)
    return out2.reshape(batch, out_dim)
